# baseline (speedup 1.0000x reference)
import functools
import math

import jax
import jax.numpy as jnp
from jax import lax
from jax.experimental import pallas as pl
from jax.experimental.pallas import tpu as pltpu
from jax.experimental.pallas import tpu_sc as plsc

D_MODEL = 64
SCALE = math.sqrt(D_MODEL)

NC = 2
NS = 16
NW = NC * NS

CHUNK = 128
B_TOTAL = 4096 * 200
B_PER_W = B_TOTAL // NW
CHUNKS_PER_W = B_PER_W // CHUNK


def _emb_kernel(x_hbm, table_hbm, out_hbm, idx_v, rows_v, sem):
    wid = lax.axis_index("s") * NC + lax.axis_index("c")
    base = wid * B_PER_W
    pltpu.sync_copy(x_hbm.at[wid], idx_v)

    def chunk_body(g, carry):
        pltpu.async_copy(table_hbm.at[idx_v.at[g]], rows_v, sem).wait()

        def scale_row(r, carry2):
            for c in range(D_MODEL // 16):
                rows_v[r, pl.ds(c * 16, 16)] = (
                    rows_v[r, pl.ds(c * 16, 16)] * SCALE
                )
            return carry2

        lax.fori_loop(0, CHUNK, scale_row, 0, unroll=4)
        pltpu.sync_copy(rows_v, out_hbm.at[pl.ds(base + g * CHUNK, CHUNK)])
        return carry

    lax.fori_loop(0, CHUNKS_PER_W, chunk_body, 0)


@jax.jit
def _embedding(x_flat, table):
    mesh = plsc.VectorSubcoreMesh(core_axis_name="c", subcore_axis_name="s")
    kfn = functools.partial(
        pl.kernel,
        mesh=mesh,
        out_type=jax.ShapeDtypeStruct((B_TOTAL, D_MODEL), jnp.float32),
        scratch_types=[
            pltpu.VMEM((CHUNKS_PER_W, CHUNK), jnp.int32),
            pltpu.VMEM((CHUNK, D_MODEL), jnp.float32),
            pltpu.SemaphoreType.DMA,
        ],
        compiler_params=pltpu.CompilerParams(use_tc_tiling_on_sc=False),
    )(_emb_kernel)
    return kfn(x_flat, table)


def kernel(x, table):
    x_flat = x.astype(jnp.int32).reshape(NW, CHUNKS_PER_W, CHUNK)
    out = _embedding(x_flat, table)
    return out.reshape(x.shape[0], x.shape[1], D_MODEL)
